# baseline (speedup 1.0000x reference)
import functools

import jax
import jax.numpy as jnp
from jax import lax
from jax.experimental import pallas as pl
from jax.experimental.pallas import tpu as pltpu
from jax.experimental.pallas import tpu_sc as plsc

TEMP = 2.718281828459045
NEG_INF = -1e30


def _logits_body(x_ref, Wg_ref, bg_ref, lt_ref):
    logits = jax.lax.dot_general(
        x_ref[...], Wg_ref[...], (((1,), (1,)), ((), ())),
        preferred_element_type=jnp.float32)
    lt_ref[...] = jnp.transpose(logits) + bg_ref[...]


def _gate_sc_body(lt_hbm, wt_hbm, lt_v, wt_v, *, num_cores, chunk, E, na):
    wid = lax.axis_index("s") * num_cores + lax.axis_index("c")
    base = wid * chunk
    pltpu.sync_copy(lt_hbm.at[:, pl.ds(base, chunk)], lt_v)
    for g in range(chunk // 16):
        sl = pl.ds(g * 16, 16)
        scaled = [lt_v[e, sl] / TEMP for e in range(E)]
        m = scaled[0]
        for e in range(1, E):
            m = jnp.maximum(m, scaled[e])
        ex = [jnp.exp(s - m) for s in scaled]
        ssum = ex[0]
        for e in range(1, E):
            ssum = ssum + ex[e]
        p = [v / ssum for v in ex]
        work = list(p)
        mask = [jnp.zeros((16,), jnp.float32) for _ in range(E)]
        big = jnp.full((16,), E, jnp.int32)
        for _ in range(na):
            mx = work[0]
            for e in range(1, E):
                mx = jnp.maximum(mx, work[e])
            sel = big
            for e in range(E):
                cand = jnp.where(work[e] == mx,
                                 jnp.full((16,), e, jnp.int32), big)
                sel = jnp.minimum(sel, cand)
            for e in range(E):
                hit = sel == jnp.full((16,), e, jnp.int32)
                mask[e] = jnp.where(hit, jnp.full((16,), 1.0, jnp.float32),
                                    mask[e])
                work[e] = jnp.where(hit, jnp.full((16,), NEG_INF, jnp.float32),
                                    work[e])
        wm = [p[e] * mask[e] for e in range(E)]
        s2 = wm[0]
        for e in range(1, E):
            s2 = s2 + wm[e]
        s2 = s2 + 1e-8
        for e in range(E):
            wt_v[e, sl] = wm[e] / s2
    pltpu.sync_copy(wt_v, wt_hbm.at[:, pl.ds(base, chunk)])


def _moe_body(x_ref, wt_ref, b1_ref, b2_ref, W1_ref, W2_ref,
              o_ref, w_ref, xb_ref, *, bt):
    e = pl.program_id(1)
    E = wt_ref.shape[0]

    @pl.when(e == 0)
    def _prep():
        xb_ref[...] = x_ref[...].astype(jnp.bfloat16)
        w = jnp.transpose(wt_ref[...])
        w_ref[...] = w
        o_ref[...] = jax.lax.dot_general(
            w, b2_ref[...], (((1,), (0,)), ((), ())),
            preferred_element_type=jnp.float32)

    xb = xb_ref[...]
    h = jax.lax.dot_general(
        xb, W1_ref[0].astype(jnp.bfloat16), (((1,), (1,)), ((), ())),
        preferred_element_type=jnp.float32)
    h = jnp.maximum(h + b1_ref[0], 0.0).astype(jnp.bfloat16)
    o = jax.lax.dot_general(
        h, W2_ref[0].astype(jnp.bfloat16), (((1,), (1,)), ((), ())),
        preferred_element_type=jnp.float32)
    lane = jax.lax.broadcasted_iota(jnp.int32, (bt, E), 1)
    w_col = jnp.sum(
        jnp.where(lane == e, w_ref[...], 0.0), axis=-1, keepdims=True)
    o_ref[...] += w_col * o


def kernel(x, Wg, bg, W1, b1, W2, b2):
    T, D = x.shape
    E, H, _ = W1.shape
    O = W2.shape[1]
    na = max(1, int(E * 0.7))
    bt = min(1024, T)

    lt = pl.pallas_call(
        _logits_body,
        grid=(T // bt,),
        in_specs=[
            pl.BlockSpec((bt, D), lambda t: (t, 0)),
            pl.BlockSpec((E, D), lambda t: (0, 0)),
            pl.BlockSpec((E, 1), lambda t: (0, 0)),
        ],
        out_specs=pl.BlockSpec((E, bt), lambda t: (0, t)),
        out_shape=jax.ShapeDtypeStruct((E, T), jnp.float32),
    )(x, Wg, bg.reshape(E, 1))

    info = plsc.get_sparse_core_info()
    nw = info.num_cores * info.num_subcores
    chunk = T // nw
    gate = functools.partial(
        _gate_sc_body, num_cores=info.num_cores, chunk=chunk, E=E, na=na)
    wt = pl.kernel(
        gate,
        out_type=jax.ShapeDtypeStruct((E, T), jnp.float32),
        mesh=plsc.VectorSubcoreMesh(core_axis_name="c", subcore_axis_name="s"),
        scratch_types=[pltpu.VMEM((E, chunk), jnp.float32),
                       pltpu.VMEM((E, chunk), jnp.float32)],
    )(lt)

    body = functools.partial(_moe_body, bt=bt)
    out = pl.pallas_call(
        body,
        grid=(T // bt, E),
        in_specs=[
            pl.BlockSpec((bt, D), lambda t, e: (t, 0)),
            pl.BlockSpec((E, bt), lambda t, e: (0, t)),
            pl.BlockSpec((1, 1, H), lambda t, e: (e, 0, 0)),
            pl.BlockSpec((E, O), lambda t, e: (0, 0)),
            pl.BlockSpec((1, H, D), lambda t, e: (e, 0, 0)),
            pl.BlockSpec((1, O, H), lambda t, e: (e, 0, 0)),
        ],
        out_specs=pl.BlockSpec((bt, O), lambda t, e: (t, 0)),
        out_shape=jax.ShapeDtypeStruct((T, O), jnp.float32),
        scratch_shapes=[pltpu.VMEM((bt, E), jnp.float32),
                        pltpu.VMEM((bt, D), jnp.bfloat16)],
        compiler_params=pltpu.CompilerParams(
            dimension_semantics=("parallel", "arbitrary")),
    )(x, wt, b1.reshape(E, 1, H), b2, W1, W2)
    return out


# speedup vs baseline: 1.1253x; 1.1253x over previous
import functools

import jax
import jax.numpy as jnp
from jax.experimental import pallas as pl
from jax.experimental.pallas import tpu as pltpu

TEMP = 2.718281828459045
NEG_INF = -1e30
H_CHUNKS = 1


def _moe_body(x_ref, Wg_ref, bg_ref, b1_ref, b2_ref, W1_ref, W2_ref,
              o_ref, w_ref, xb_ref, *, na, bt):
    e = pl.program_id(1)
    E = Wg_ref.shape[0]

    @pl.when(e == 0)
    def _gate():
        x = x_ref[...]
        xb_ref[...] = x.astype(jnp.bfloat16)
        logits = jax.lax.dot_general(
            x, Wg_ref[...], (((1,), (1,)), ((), ())),
            preferred_element_type=jnp.float32)
        logits_t = jnp.transpose(logits) + bg_ref[...]
        scaled = logits_t / TEMP
        m = jnp.max(scaled, axis=0, keepdims=True)
        ex = jnp.exp(scaled - m)
        p = ex / jnp.sum(ex, axis=0, keepdims=True)
        iota = jax.lax.broadcasted_iota(jnp.int32, (E, bt), 0)
        work = p
        mask = jnp.zeros((E, bt), dtype=jnp.float32)
        for _ in range(na):
            mx = jnp.max(work, axis=0, keepdims=True)
            cand = jnp.where(work == mx, iota, E)
            sel = jnp.min(cand, axis=0, keepdims=True)
            onehot = (iota == sel).astype(jnp.float32)
            mask = mask + onehot
            work = jnp.where(onehot > 0, NEG_INF, work)
        w_t = p * mask
        w_t = w_t / (jnp.sum(w_t, axis=0, keepdims=True) + 1e-8)
        w = jnp.transpose(w_t)
        w_ref[...] = w
        o_ref[...] = jax.lax.dot_general(
            w, b2_ref[...], (((1,), (0,)), ((), ())),
            preferred_element_type=jnp.float32)

    xb = xb_ref[...]
    H = W1_ref.shape[1]
    hc = H // H_CHUNKS
    o_acc = None
    for k in range(H_CHUNKS):
        w1k = W1_ref[0, k * hc:(k + 1) * hc, :].astype(jnp.bfloat16)
        hk = jax.lax.dot_general(
            xb, w1k, (((1,), (1,)), ((), ())),
            preferred_element_type=jnp.float32)
        hk = jnp.maximum(hk + b1_ref[0, 0, k * hc:(k + 1) * hc], 0.0)
        hk = hk.astype(jnp.bfloat16)
        w2k = W2_ref[0, :, k * hc:(k + 1) * hc].astype(jnp.bfloat16)
        ok = jax.lax.dot_general(
            hk, w2k, (((1,), (1,)), ((), ())),
            preferred_element_type=jnp.float32)
        o_acc = ok if o_acc is None else o_acc + ok
    lane = jax.lax.broadcasted_iota(jnp.int32, (bt, E), 1)
    w_col = jnp.sum(
        jnp.where(lane == e, w_ref[...], 0.0), axis=-1, keepdims=True)
    o_ref[...] += w_col * o_acc


def kernel(x, Wg, bg, W1, b1, W2, b2):
    T, D = x.shape
    E, H, _ = W1.shape
    O = W2.shape[1]
    na = max(1, int(E * 0.7))
    bt = min(1024, T)
    grid = (T // bt, E)

    body = functools.partial(_moe_body, na=na, bt=bt)
    out = pl.pallas_call(
        body,
        grid=grid,
        in_specs=[
            pl.BlockSpec((bt, D), lambda t, e: (t, 0)),
            pl.BlockSpec((E, D), lambda t, e: (0, 0)),
            pl.BlockSpec((E, 1), lambda t, e: (0, 0)),
            pl.BlockSpec((1, 1, H), lambda t, e: (e, 0, 0)),
            pl.BlockSpec((E, O), lambda t, e: (0, 0)),
            pl.BlockSpec((1, H, D), lambda t, e: (e, 0, 0)),
            pl.BlockSpec((1, O, H), lambda t, e: (e, 0, 0)),
        ],
        out_specs=pl.BlockSpec((bt, O), lambda t, e: (t, 0)),
        out_shape=jax.ShapeDtypeStruct((T, O), jnp.float32),
        scratch_shapes=[pltpu.VMEM((bt, E), jnp.float32),
                        pltpu.VMEM((bt, D), jnp.bfloat16)],
        compiler_params=pltpu.CompilerParams(
            dimension_semantics=("parallel", "arbitrary")),
    )(x, Wg, bg.reshape(E, 1), b1.reshape(E, 1, H), b2, W1, W2)
    return out


# speedup vs baseline: 1.1408x; 1.0138x over previous
import functools

import jax
import jax.numpy as jnp
from jax.experimental import pallas as pl
from jax.experimental.pallas import tpu as pltpu

TEMP = 2.718281828459045
NEG_INF = -1e30
H_CHUNKS = 1


def _moe_body(x_ref, Wg_ref, bg_ref, b1_ref, b2_ref, W1_ref, W2_ref,
              o_ref, w_ref, xb_ref, *, na, bt):
    e = pl.program_id(1)
    E = Wg_ref.shape[0]

    @pl.when(e == 0)
    def _gate():
        x = x_ref[...]
        xb_ref[...] = x.astype(jnp.bfloat16)
        logits = jax.lax.dot_general(
            x, Wg_ref[...], (((1,), (1,)), ((), ())),
            preferred_element_type=jnp.float32)
        logits_t = jnp.transpose(logits) + bg_ref[...]
        scaled = logits_t / TEMP
        m = jnp.max(scaled, axis=0, keepdims=True)
        ex = jnp.exp(scaled - m)
        p = ex / jnp.sum(ex, axis=0, keepdims=True)
        iota = jax.lax.broadcasted_iota(jnp.int32, (E, bt), 0)
        work = p
        mask = jnp.zeros((E, bt), dtype=jnp.float32)
        for _ in range(na):
            mx = jnp.max(work, axis=0, keepdims=True)
            cand = jnp.where(work == mx, iota, E)
            sel = jnp.min(cand, axis=0, keepdims=True)
            onehot = (iota == sel).astype(jnp.float32)
            mask = mask + onehot
            work = jnp.where(onehot > 0, NEG_INF, work)
        w_t = p * mask
        w_t = w_t / (jnp.sum(w_t, axis=0, keepdims=True) + 1e-8)
        w = jnp.transpose(w_t)
        w_ref[...] = w
        o_ref[...] = jax.lax.dot_general(
            w, b2_ref[...], (((1,), (0,)), ((), ())),
            preferred_element_type=jnp.float32)

    xb = xb_ref[...]
    H = W1_ref.shape[1]
    hc = H // H_CHUNKS
    o_acc = None
    for k in range(H_CHUNKS):
        w1k = W1_ref[0, k * hc:(k + 1) * hc, :].astype(jnp.bfloat16)
        hk = jax.lax.dot_general(
            xb, w1k, (((1,), (1,)), ((), ())),
            preferred_element_type=jnp.float32)
        hk = jnp.maximum(hk + b1_ref[0, 0, k * hc:(k + 1) * hc], 0.0)
        hk = hk.astype(jnp.bfloat16)
        w2k = W2_ref[0, :, k * hc:(k + 1) * hc].astype(jnp.bfloat16)
        ok = jax.lax.dot_general(
            hk, w2k, (((1,), (1,)), ((), ())),
            preferred_element_type=jnp.float32)
        o_acc = ok if o_acc is None else o_acc + ok
    lane = jax.lax.broadcasted_iota(jnp.int32, (bt, E), 1)
    w_col = jnp.sum(
        jnp.where(lane == e, w_ref[...], 0.0), axis=-1, keepdims=True)
    o_ref[...] += w_col * o_acc


def kernel(x, Wg, bg, W1, b1, W2, b2):
    T, D = x.shape
    E, H, _ = W1.shape
    O = W2.shape[1]
    na = max(1, int(E * 0.7))
    bt = min(2048, T)
    grid = (T // bt, E)

    body = functools.partial(_moe_body, na=na, bt=bt)
    out = pl.pallas_call(
        body,
        grid=grid,
        in_specs=[
            pl.BlockSpec((bt, D), lambda t, e: (t, 0)),
            pl.BlockSpec((E, D), lambda t, e: (0, 0)),
            pl.BlockSpec((E, 1), lambda t, e: (0, 0)),
            pl.BlockSpec((1, 1, H), lambda t, e: (e, 0, 0)),
            pl.BlockSpec((E, O), lambda t, e: (0, 0)),
            pl.BlockSpec((1, H, D), lambda t, e: (e, 0, 0)),
            pl.BlockSpec((1, O, H), lambda t, e: (e, 0, 0)),
        ],
        out_specs=pl.BlockSpec((bt, O), lambda t, e: (t, 0)),
        out_shape=jax.ShapeDtypeStruct((T, O), jnp.float32),
        scratch_shapes=[pltpu.VMEM((bt, E), jnp.float32),
                        pltpu.VMEM((bt, D), jnp.bfloat16)],
        compiler_params=pltpu.CompilerParams(
            dimension_semantics=("parallel", "arbitrary"),
            vmem_limit_bytes=100 * 1024 * 1024),
    )(x, Wg, bg.reshape(E, 1), b1.reshape(E, 1, H), b2, W1, W2)
    return out
